# baseline (speedup 1.0000x reference)
import jax
import jax.numpy as jnp
from jax import lax
from jax.experimental import pallas as pl
from jax.experimental.pallas import tpu as pltpu
from jax.experimental.pallas import tpu_sc as plsc

_NF, _V, _D, _B = 26, 100000, 32, 16384
_NC = 2
_NW = 32
_NP = _NF * _D
_PPW = _NP // _NW
_BCH = 2048
_NBCH = _B // _BCH
_NT = _PPW * _NBCH
_UNROLL = 32


def _body(tab_hbm, idx_hbm, out_hbm, plane_v, idx_v, res0_v, res1_v, ws0, ws1):
    wid = lax.axis_index("s") * _NC + lax.axis_index("c")
    res = (res0_v, res1_v)
    wsem = (ws0, ws1)

    def chunk(t, f_prev, s, drain):
        k = t // _NBCH
        c = t % _NBCH
        p = wid * _PPW + k
        f = p // _D
        d = p % _D

        @pl.when(c == 0)
        def _():
            pltpu.sync_copy(tab_hbm.at[f, d], plane_v)

        @pl.when(f != f_prev)
        def _():
            pltpu.sync_copy(idx_hbm.at[f], idx_v)

        if drain:
            pltpu.make_async_copy(
                res[s], out_hbm.at[p, pl.ds(c * _BCH, _BCH)], wsem[s]
            ).wait()

        @plsc.parallel_loop(0, _BCH, 16, unroll=_UNROLL)
        def _(i):
            iv = idx_v[pl.ds(c * _BCH + i, 16)]
            res[s][pl.ds(i, 16)] = plsc.load_gather(plane_v, [iv])
        pltpu.async_copy(res[s], out_hbm.at[p, pl.ds(c * _BCH, _BCH)], wsem[s])
        return f

    f_prev = chunk(0, jnp.int32(-1), 0, False)
    f_prev = chunk(1, f_prev, 1, False)

    def pair(g, f_prev):
        t = g * 2
        f_prev = chunk(t, f_prev, 0, True)
        f_prev = chunk(t + 1, f_prev, 1, True)
        return f_prev

    lax.fori_loop(1, _NT // 2, pair, f_prev)

    for s in range(2):
        pltpu.make_async_copy(
            res[s], out_hbm.at[_NP - 1, pl.ds(0, _BCH)], wsem[s]
        ).wait()


@jax.jit
def kernel(x_cat, tables):
    tab_t = jnp.transpose(tables, (0, 2, 1))
    idx_t = jnp.transpose(x_cat, (1, 0))

    mesh = plsc.VectorSubcoreMesh(core_axis_name="c", subcore_axis_name="s")
    run = pl.kernel(
        _body,
        out_type=jax.ShapeDtypeStruct((_NP, _B), jnp.float32),
        mesh=mesh,
        scratch_types=[
            pltpu.VMEM((_V,), jnp.float32),
            pltpu.VMEM((_B,), jnp.int32),
            pltpu.VMEM((_BCH,), jnp.float32),
            pltpu.VMEM((_BCH,), jnp.float32),
            pltpu.SemaphoreType.DMA,
            pltpu.SemaphoreType.DMA,
        ],
        compiler_params=pltpu.CompilerParams(needs_layout_passes=False),
    )
    out_t = run(tab_t, idx_t)
    return jnp.transpose(out_t, (1, 0))


# speedup vs baseline: 1.0039x; 1.0039x over previous
import jax
import jax.numpy as jnp
from jax import lax
from jax.experimental import pallas as pl
from jax.experimental.pallas import tpu as pltpu
from jax.experimental.pallas import tpu_sc as plsc

_NF, _V, _D, _B = 26, 100000, 32, 16384
_NC = 2
_NW = 32
_NP = _NF * _D
_PPW = _NP // _NW
_BCH = 4096
_NBCH = _B // _BCH
_NT = _PPW * _NBCH
_UNROLL = 16


def _body(tab_hbm, idx_hbm, out_hbm, plane_v, idx_v, res0_v, res1_v, ws0, ws1):
    wid = lax.axis_index("s") * _NC + lax.axis_index("c")
    res = (res0_v, res1_v)
    wsem = (ws0, ws1)

    def chunk(t, f_prev, s, drain):
        k = t // _NBCH
        c = t % _NBCH
        p = wid * _PPW + k
        f = p // _D
        d = p % _D

        @pl.when(c == 0)
        def _():
            pltpu.sync_copy(tab_hbm.at[f, d], plane_v)

        @pl.when(f != f_prev)
        def _():
            pltpu.sync_copy(idx_hbm.at[f], idx_v)

        if drain:
            pltpu.make_async_copy(
                res[s], out_hbm.at[p, pl.ds(c * _BCH, _BCH)], wsem[s]
            ).wait()

        @plsc.parallel_loop(0, _BCH, 16, unroll=_UNROLL)
        def _(i):
            iv = idx_v[pl.ds(c * _BCH + i, 16)]
            res[s][pl.ds(i, 16)] = plsc.load_gather(plane_v, [iv])
        pltpu.async_copy(res[s], out_hbm.at[p, pl.ds(c * _BCH, _BCH)], wsem[s])
        return f

    f_prev = chunk(0, jnp.int32(-1), 0, False)
    f_prev = chunk(1, f_prev, 1, False)

    def pair(g, f_prev):
        t = g * 2
        f_prev = chunk(t, f_prev, 0, True)
        f_prev = chunk(t + 1, f_prev, 1, True)
        return f_prev

    lax.fori_loop(1, _NT // 2, pair, f_prev)

    for s in range(2):
        pltpu.make_async_copy(
            res[s], out_hbm.at[_NP - 1, pl.ds(0, _BCH)], wsem[s]
        ).wait()


@jax.jit
def kernel(x_cat, tables):
    tab_t = jnp.transpose(tables, (0, 2, 1))
    idx_t = jnp.transpose(x_cat, (1, 0))

    mesh = plsc.VectorSubcoreMesh(core_axis_name="c", subcore_axis_name="s")
    run = pl.kernel(
        _body,
        out_type=jax.ShapeDtypeStruct((_NP, _B), jnp.float32),
        mesh=mesh,
        scratch_types=[
            pltpu.VMEM((_V,), jnp.float32),
            pltpu.VMEM((_B,), jnp.int32),
            pltpu.VMEM((_BCH,), jnp.float32),
            pltpu.VMEM((_BCH,), jnp.float32),
            pltpu.SemaphoreType.DMA,
            pltpu.SemaphoreType.DMA,
        ],
        compiler_params=pltpu.CompilerParams(needs_layout_passes=False),
    )
    out_t = run(tab_t, idx_t)
    return jnp.transpose(out_t, (1, 0))


# speedup vs baseline: 1.0046x; 1.0007x over previous
import jax
import jax.numpy as jnp
from jax import lax
from jax.experimental import pallas as pl
from jax.experimental.pallas import tpu as pltpu
from jax.experimental.pallas import tpu_sc as plsc

_NF, _V, _D, _B = 26, 100000, 32, 16384
_NC = 2
_NW = 32
_NP = _NF * _D
_PPW = _NP // _NW
_BCH = 4096
_NBCH = _B // _BCH
_NT = _PPW * _NBCH
_UNROLL = 16


def _body(tab_hbm, idx_hbm, out_hbm, plane_v, idx_v, res0_v, res1_v, ws0, ws1):
    wid = lax.axis_index("s") * _NC + lax.axis_index("c")
    res = (res0_v, res1_v)
    wsem = (ws0, ws1)

    def chunk(t, f_prev, s, drain):
        k = t // _NBCH
        c = t % _NBCH
        p = wid * _PPW + k
        f = p // _D
        d = p % _D

        @pl.when(c == 0)
        def _():
            pltpu.sync_copy(tab_hbm.at[f, d], plane_v)

        @pl.when(f != f_prev)
        def _():
            pltpu.sync_copy(idx_hbm.at[f], idx_v)

        if drain:
            pltpu.make_async_copy(
                res[s], out_hbm.at[p, pl.ds(c * _BCH, _BCH)], wsem[s]
            ).wait()

        @plsc.parallel_loop(0, _BCH, 16, unroll=_UNROLL)
        def _(i):
            iv = idx_v[pl.ds(c * _BCH + i, 16)]
            res[s][pl.ds(i, 16)] = plsc.load_gather(plane_v, [iv])
        pltpu.async_copy(res[s], out_hbm.at[p, pl.ds(c * _BCH, _BCH)], wsem[s])
        return f

    f_prev = chunk(0, jnp.int32(-1), 0, False)
    f_prev = chunk(1, f_prev, 1, False)

    def pair(g, f_prev):
        t = g * 2
        f_prev = chunk(t, f_prev, 0, True)
        f_prev = chunk(t + 1, f_prev, 1, True)
        return f_prev

    lax.fori_loop(1, _NT // 2, pair, f_prev)

    for s in range(2):
        pltpu.make_async_copy(
            res[s], out_hbm.at[_NP - 1, pl.ds(0, _BCH)], wsem[s]
        ).wait()


@jax.jit
def kernel(x_cat, tables):
    tab_t = jnp.transpose(tables, (0, 2, 1))
    idx_t = jnp.transpose(x_cat.astype(jnp.int32), (1, 0))

    mesh = plsc.VectorSubcoreMesh(core_axis_name="c", subcore_axis_name="s")
    run = pl.kernel(
        _body,
        out_type=jax.ShapeDtypeStruct((_NP, _B), jnp.float32),
        mesh=mesh,
        scratch_types=[
            pltpu.VMEM((_V,), jnp.float32),
            pltpu.VMEM((_B,), jnp.int32),
            pltpu.VMEM((_BCH,), jnp.float32),
            pltpu.VMEM((_BCH,), jnp.float32),
            pltpu.SemaphoreType.DMA,
            pltpu.SemaphoreType.DMA,
        ],
        compiler_params=pltpu.CompilerParams(needs_layout_passes=False),
    )
    out_t = run(tab_t, idx_t)
    return jnp.transpose(out_t, (1, 0))
